# initial kernel scaffold (unmeasured)
import jax
import jax.numpy as jnp
from jax import lax
from jax.experimental import pallas as pl
from jax.experimental.pallas import tpu as pltpu

N_DEV = 4
TOK = 2048
D = 1024
E_LOCAL = 8
N_EXP = 32


def kernel(x, router_W, route_idx, expert_W):
    def body(x_ref, rw_ref, idx_ref, ew_ref, out_ref,
             meta_local, ag_x, ag_meta, p_acc, rs_buf, w_buf,
             ag_x_send, ag_x_recv, ag_m_send, ag_m_recv,
             rs_send, rs_recv, w_sem, credit_ag, credit_rs):
        me = lax.axis_index("i")
        left = lax.rem(me + N_DEV - 1, N_DEV)
        right = lax.rem(me + 1, N_DEV)

        barrier = pltpu.get_barrier_semaphore()
        for nbr in (left, right):
            pl.semaphore_signal(barrier, inc=1, device_id=(nbr,),
                                device_id_type=pl.DeviceIdType.MESH)
        pl.semaphore_wait(barrier, 2)

        x_val = x_ref[...]
        scores = jnp.dot(x_val, rw_ref[...], preferred_element_type=jnp.float32)
        smax = jnp.max(scores, axis=1, keepdims=True)
        probs = jnp.exp(scores - smax)
        probs = probs / jnp.sum(probs, axis=1, keepdims=True)
        idx = idx_ref[...]
        iota = lax.broadcasted_iota(jnp.int32, (TOK, N_EXP), 1)
        g0 = jnp.sum(jnp.where(iota == idx[:, 0:1], probs, 0.0),
                     axis=1, keepdims=True)
        g1 = jnp.sum(jnp.where(iota == idx[:, 1:2], probs, 0.0),
                     axis=1, keepdims=True)
        gs = g0 + g1
        meta_local[...] = jnp.concatenate(
            [g0 / gs, g1 / gs,
             idx[:, 0:1].astype(jnp.float32),
             idx[:, 1:2].astype(jnp.float32)],
            axis=1,
        )

        def accumulate(x_sref, meta_ref, acc_ref):
            meta = meta_ref[...]
            mg0 = meta[:, 0:1]
            mg1 = meta[:, 1:2]
            mi0 = meta[:, 2:3]
            mi1 = meta[:, 3:4]
            acc_ref[...] = jnp.zeros((TOK, D), jnp.float32)

            def exp_body(e, carry):
                cp = pltpu.make_async_copy(ew_ref.at[e], w_buf, w_sem)
                cp.start()
                gid = (me * E_LOCAL + e).astype(jnp.float32)
                gate = (jnp.where(mi0 == gid, mg0, 0.0)
                        + jnp.where(mi1 == gid, mg1, 0.0))
                xm = x_sref[...] * gate
                cp.wait()
                acc_ref[...] = acc_ref[...] + jnp.dot(
                    xm, w_buf[...], preferred_element_type=jnp.float32)
                return carry

            lax.fori_loop(0, E_LOCAL, exp_body, 0)

        def rcopy(src, dst, ssem, rsem):
            return pltpu.make_async_remote_copy(
                src_ref=src, dst_ref=dst, send_sem=ssem, recv_sem=rsem,
                device_id=(right,), device_id_type=pl.DeviceIdType.MESH)

        x0 = rcopy(x_ref, ag_x.at[0], ag_x_send.at[0], ag_x_recv.at[0])
        m0 = rcopy(meta_local, ag_meta.at[0], ag_m_send.at[0], ag_m_recv.at[0])
        x0.start()
        m0.start()
        accumulate(x_ref, meta_local, out_ref)
        x0.wait()
        m0.wait()

        x1 = rcopy(ag_x.at[0], ag_x.at[1], ag_x_send.at[1], ag_x_recv.at[1])
        m1 = rcopy(ag_meta.at[0], ag_meta.at[1], ag_m_send.at[1], ag_m_recv.at[1])
        x1.start()
        m1.start()
        accumulate(ag_x.at[0], ag_meta.at[0], p_acc)
        x1.wait()
        m1.wait()
        pl.semaphore_signal(credit_ag, inc=1, device_id=(left,),
                            device_id_type=pl.DeviceIdType.MESH)

        r0 = rcopy(p_acc, rs_buf.at[0], rs_send.at[0], rs_recv.at[0])
        r0.start()
        pl.semaphore_wait(credit_ag, 1)
        x2 = rcopy(ag_x.at[1], ag_x.at[0], ag_x_send.at[2], ag_x_recv.at[2])
        m2 = rcopy(ag_meta.at[1], ag_meta.at[0], ag_m_send.at[2], ag_m_recv.at[2])
        x2.start()
        m2.start()
        r0.wait()
        x2.wait()
        m2.wait()

        accumulate(ag_x.at[1], ag_meta.at[1], p_acc)
        p_acc[...] = p_acc[...] + rs_buf[0]
        pl.semaphore_signal(credit_rs, inc=1, device_id=(left,),
                            device_id_type=pl.DeviceIdType.MESH)

        r1 = rcopy(p_acc, rs_buf.at[1], rs_send.at[1], rs_recv.at[1])
        r1.start()
        r1.wait()

        accumulate(ag_x.at[0], ag_meta.at[0], p_acc)
        p_acc[...] = p_acc[...] + rs_buf[1]

        pl.semaphore_wait(credit_rs, 1)
        r2 = rcopy(p_acc, rs_buf.at[0], rs_send.at[2], rs_recv.at[2])
        r2.start()
        r2.wait()
        out_ref[...] = out_ref[...] + rs_buf[0]

    return pl.pallas_call(
        body,
        out_shape=jax.ShapeDtypeStruct((TOK, D), jnp.float32),
        in_specs=[
            pl.BlockSpec(memory_space=pltpu.MemorySpace.VMEM),
            pl.BlockSpec(memory_space=pltpu.MemorySpace.VMEM),
            pl.BlockSpec(memory_space=pltpu.MemorySpace.VMEM),
            pl.BlockSpec(memory_space=pltpu.MemorySpace.HBM),
        ],
        out_specs=pl.BlockSpec(memory_space=pltpu.MemorySpace.VMEM),
        scratch_shapes=[
            pltpu.VMEM((TOK, 4), jnp.float32),
            pltpu.VMEM((2, TOK, D), jnp.float32),
            pltpu.VMEM((2, TOK, 4), jnp.float32),
            pltpu.VMEM((TOK, D), jnp.float32),
            pltpu.VMEM((2, TOK, D), jnp.float32),
            pltpu.VMEM((D, D), jnp.float32),
            pltpu.SemaphoreType.DMA((3,)),
            pltpu.SemaphoreType.DMA((3,)),
            pltpu.SemaphoreType.DMA((3,)),
            pltpu.SemaphoreType.DMA((3,)),
            pltpu.SemaphoreType.DMA((3,)),
            pltpu.SemaphoreType.DMA((3,)),
            pltpu.SemaphoreType.DMA,
            pltpu.SemaphoreType.REGULAR,
            pltpu.SemaphoreType.REGULAR,
        ],
        compiler_params=pltpu.CompilerParams(
            collective_id=0,
            vmem_limit_bytes=64 * 1024 * 1024,
            has_side_effects=True,
        ),
    )(x, router_W, route_idx, expert_W)


# baseline (device time: 460949 ns/iter reference)
import jax
import jax.numpy as jnp
from jax import lax
from jax.experimental import pallas as pl
from jax.experimental.pallas import tpu as pltpu

N_DEV = 4
TOK = 2048
D = 1024
E_LOCAL = 8
N_EXP = 32
BLK = 512
NBLK = TOK // BLK


def kernel(x, router_W, route_idx, expert_W):
    def body(x_ref, rw_ref, idx_ref, ew_ref, out_ref,
             x_bf, meta_local, ag_x, ag_meta, p_acc, rs_send, rs_buf, w_buf,
             gate_ref,
             ag_x_send, ag_x_recv, ag_m_send, ag_m_recv,
             rs_send_sem, rs_recv_sem, w_sem, credit_ag, credit_rs):
        me = lax.axis_index("i")
        left = lax.rem(me + N_DEV - 1, N_DEV)
        right = lax.rem(me + 1, N_DEV)

        barrier = pltpu.get_barrier_semaphore()
        for nbr in (left, right):
            pl.semaphore_signal(barrier, inc=1, device_id=(nbr,),
                                device_id_type=pl.DeviceIdType.MESH)
        pl.semaphore_wait(barrier, 2)

        scores = jnp.dot(x_ref[...], rw_ref[...],
                         preferred_element_type=jnp.float32)
        smax = jnp.max(scores, axis=1, keepdims=True)
        probs = jnp.exp(scores - smax)
        probs = probs / jnp.sum(probs, axis=1, keepdims=True)
        idx = idx_ref[...]
        iota = lax.broadcasted_iota(jnp.int32, (TOK, N_EXP), 1)
        g0 = jnp.sum(jnp.where(iota == idx[:, 0:1], probs, 0.0),
                     axis=1, keepdims=True)
        g1 = jnp.sum(jnp.where(iota == idx[:, 1:2], probs, 0.0),
                     axis=1, keepdims=True)
        gs = g0 + g1
        my_g0 = g0 / gs
        my_g1 = g1 / gs
        my_i0 = idx[:, 0:1].astype(jnp.float32)
        my_i1 = idx[:, 1:2].astype(jnp.float32)
        meta_local[0:4, :] = jnp.concatenate(
            [my_g0, my_g1, my_i0, my_i1], axis=1).T

        def _xbf_body(b, c):
            sl = pl.ds(b * BLK, BLK)
            x_bf[sl, :] = x_ref[sl, :].astype(jnp.bfloat16)
            return c
        lax.fori_loop(0, NBLK, _xbf_body, 0)

        def meta_cols(meta_ref):
            mt = meta_ref[...].T
            return mt[:, 2:3], mt[:, 3:4], mt[:, 0:1], mt[:, 1:2]

        def accumulate(x_src, i0, i1, gg0, gg1, acc_ref):
            acc_ref[...] = jnp.zeros((TOK, D), jnp.float32)
            me_f = me.astype(jnp.float32)

            for e in range(E_LOCAL):
                cp = pltpu.make_async_copy(ew_ref.at[e], w_buf, w_sem)
                cp.start()
                gid = me_f * E_LOCAL + e
                gate_ref[...] = (jnp.where(i0 == gid, gg0, 0.0)
                                 + jnp.where(i1 == gid, gg1, 0.0))
                cp.wait()

                def blk_body(b, c):
                    sl = pl.ds(b * BLK, BLK)
                    xm = (x_src[sl, :].astype(jnp.float32)
                          * gate_ref[sl, :])
                    acc_ref[sl, :] = acc_ref[sl, :] + jnp.dot(
                        xm, w_buf[...], preferred_element_type=jnp.float32)
                    return c

                lax.fori_loop(0, NBLK, blk_body, 0)

        def add_recv(acc_ref, slot):
            def blk_body(b, c):
                sl = pl.ds(b * BLK, BLK)
                acc_ref[sl, :] = (acc_ref[sl, :]
                                  + rs_buf[slot, sl, :].astype(jnp.float32))
                return c
            lax.fori_loop(0, NBLK, blk_body, 0)

        def to_bf16(dst_ref, src_ref):
            def blk_body(b, c):
                sl = pl.ds(b * BLK, BLK)
                dst_ref[sl, :] = src_ref[sl, :].astype(jnp.bfloat16)
                return c
            lax.fori_loop(0, NBLK, blk_body, 0)

        def rcopy(src, dst, ssem, rsem):
            return pltpu.make_async_remote_copy(
                src_ref=src, dst_ref=dst, send_sem=ssem, recv_sem=rsem,
                device_id=(right,), device_id_type=pl.DeviceIdType.MESH)

        x0 = rcopy(x_bf, ag_x.at[0], ag_x_send.at[0], ag_x_recv.at[0])
        m0 = rcopy(meta_local, ag_meta.at[0], ag_m_send.at[0], ag_m_recv.at[0])
        x0.start()
        m0.start()
        accumulate(x_ref, my_i0, my_i1, my_g0, my_g1, out_ref)
        x0.wait()
        m0.wait()

        x1 = rcopy(ag_x.at[0], ag_x.at[1], ag_x_send.at[1], ag_x_recv.at[1])
        m1 = rcopy(ag_meta.at[0], ag_meta.at[1], ag_m_send.at[1], ag_m_recv.at[1])
        x1.start()
        m1.start()
        i0, i1, gg0, gg1 = meta_cols(ag_meta.at[0])
        accumulate(ag_x.at[0], i0, i1, gg0, gg1, p_acc)
        x1.wait()
        m1.wait()
        pl.semaphore_signal(credit_ag, inc=1, device_id=(left,),
                            device_id_type=pl.DeviceIdType.MESH)

        to_bf16(rs_send, p_acc)
        r0 = rcopy(rs_send, rs_buf.at[0], rs_send_sem.at[0], rs_recv_sem.at[0])
        r0.start()
        pl.semaphore_wait(credit_ag, 1)
        x2 = rcopy(ag_x.at[1], ag_x.at[0], ag_x_send.at[2], ag_x_recv.at[2])
        m2 = rcopy(ag_meta.at[1], ag_meta.at[0], ag_m_send.at[2], ag_m_recv.at[2])
        x2.start()
        m2.start()
        r0.wait()
        x2.wait()
        m2.wait()

        i0, i1, gg0, gg1 = meta_cols(ag_meta.at[1])
        accumulate(ag_x.at[1], i0, i1, gg0, gg1, p_acc)
        add_recv(p_acc, 0)
        pl.semaphore_signal(credit_rs, inc=1, device_id=(left,),
                            device_id_type=pl.DeviceIdType.MESH)

        to_bf16(rs_send, p_acc)
        r1 = rcopy(rs_send, rs_buf.at[1], rs_send_sem.at[1], rs_recv_sem.at[1])
        r1.start()
        r1.wait()

        i0, i1, gg0, gg1 = meta_cols(ag_meta.at[0])
        accumulate(ag_x.at[0], i0, i1, gg0, gg1, p_acc)
        add_recv(p_acc, 1)

        to_bf16(rs_send, p_acc)
        pl.semaphore_wait(credit_rs, 1)
        r2 = rcopy(rs_send, rs_buf.at[0], rs_send_sem.at[2], rs_recv_sem.at[2])
        r2.start()
        r2.wait()
        add_recv(out_ref, 0)

    return pl.pallas_call(
        body,
        out_shape=jax.ShapeDtypeStruct((TOK, D), jnp.float32),
        in_specs=[
            pl.BlockSpec(memory_space=pltpu.MemorySpace.VMEM),
            pl.BlockSpec(memory_space=pltpu.MemorySpace.VMEM),
            pl.BlockSpec(memory_space=pltpu.MemorySpace.VMEM),
            pl.BlockSpec(memory_space=pltpu.MemorySpace.HBM),
        ],
        out_specs=pl.BlockSpec(memory_space=pltpu.MemorySpace.VMEM),
        scratch_shapes=[
            pltpu.VMEM((TOK, D), jnp.bfloat16),
            pltpu.VMEM((8, TOK), jnp.float32),
            pltpu.VMEM((2, TOK, D), jnp.bfloat16),
            pltpu.VMEM((2, 8, TOK), jnp.float32),
            pltpu.VMEM((TOK, D), jnp.float32),
            pltpu.VMEM((TOK, D), jnp.bfloat16),
            pltpu.VMEM((2, TOK, D), jnp.bfloat16),
            pltpu.VMEM((D, D), jnp.float32),
            pltpu.VMEM((TOK, 1), jnp.float32),
            pltpu.SemaphoreType.DMA((3,)),
            pltpu.SemaphoreType.DMA((3,)),
            pltpu.SemaphoreType.DMA((3,)),
            pltpu.SemaphoreType.DMA((3,)),
            pltpu.SemaphoreType.DMA((3,)),
            pltpu.SemaphoreType.DMA((3,)),
            pltpu.SemaphoreType.DMA,
            pltpu.SemaphoreType.REGULAR,
            pltpu.SemaphoreType.REGULAR,
        ],
        compiler_params=pltpu.CompilerParams(
            collective_id=0,
            vmem_limit_bytes=64 * 1024 * 1024,
            has_side_effects=True,
        ),
    )(x, router_W, route_idx, expert_W)


# device time: 314517 ns/iter; 1.4656x vs baseline; 1.4656x over previous
import jax
import jax.numpy as jnp
from jax import lax
from jax.experimental import pallas as pl
from jax.experimental.pallas import tpu as pltpu

N_DEV = 4
TOK = 2048
D = 1024
E_LOCAL = 8
N_EXP = 32
BLK = 512
NBLK = TOK // BLK


def kernel(x, router_W, route_idx, expert_W):
    def body(x_ref, rw_ref, idx_ref, ew_ref, out_ref,
             meta_local, ag_x, ag_meta, p_acc, rs_send, rs_buf, w_buf,
             gate_ref,
             ag_x_send, ag_x_recv, ag_m_send, ag_m_recv,
             rs_send_sem, rs_recv_sem, w_sem, credit_ag, credit_rs):
        me = lax.axis_index("i")
        left = lax.rem(me + N_DEV - 1, N_DEV)
        right = lax.rem(me + 1, N_DEV)

        barrier = pltpu.get_barrier_semaphore()
        for nbr in (left, right):
            pl.semaphore_signal(barrier, inc=1, device_id=(nbr,),
                                device_id_type=pl.DeviceIdType.MESH)
        pl.semaphore_wait(barrier, 2)

        scores = jnp.dot(x_ref[...], rw_ref[...],
                         preferred_element_type=jnp.float32)
        smax = jnp.max(scores, axis=1, keepdims=True)
        probs = jnp.exp(scores - smax)
        probs = probs / jnp.sum(probs, axis=1, keepdims=True)
        idx = idx_ref[...]
        iota = lax.broadcasted_iota(jnp.int32, (TOK, N_EXP), 1)
        g0 = jnp.sum(jnp.where(iota == idx[:, 0:1], probs, 0.0),
                     axis=1, keepdims=True)
        g1 = jnp.sum(jnp.where(iota == idx[:, 1:2], probs, 0.0),
                     axis=1, keepdims=True)
        gs = g0 + g1
        my_g0 = g0 / gs
        my_g1 = g1 / gs
        my_i0 = idx[:, 0:1].astype(jnp.float32)
        my_i1 = idx[:, 1:2].astype(jnp.float32)
        meta_local[0:4, :] = jnp.concatenate(
            [my_g0, my_g1, my_i0, my_i1], axis=1).T


        def meta_cols(meta_ref):
            mt = meta_ref[...].T
            return mt[:, 2:3], mt[:, 3:4], mt[:, 0:1], mt[:, 1:2]

        def accumulate(x_src, i0, i1, gg0, gg1, acc_ref):
            acc_ref[...] = jnp.zeros((TOK, D), jnp.float32)
            me_f = me.astype(jnp.float32)

            pltpu.make_async_copy(ew_ref.at[0], w_buf.at[0], w_sem.at[0]).start()
            for e in range(E_LOCAL):
                slot = e % 2
                if e + 1 < E_LOCAL:
                    pltpu.make_async_copy(
                        ew_ref.at[e + 1], w_buf.at[1 - slot],
                        w_sem.at[1 - slot]).start()
                gid = me_f * E_LOCAL + e
                gate_ref[...] = (jnp.where(i0 == gid, gg0, 0.0)
                                 + jnp.where(i1 == gid, gg1, 0.0))
                pltpu.make_async_copy(
                    ew_ref.at[e], w_buf.at[slot], w_sem.at[slot]).wait()

                def blk_body(b, c):
                    sl = pl.ds(b * BLK, BLK)
                    xm = (x_src[sl, :].astype(jnp.float32)
                          * gate_ref[sl, :])
                    acc_ref[sl, :] = acc_ref[sl, :] + jnp.dot(
                        xm, w_buf[slot, :, :],
                        preferred_element_type=jnp.float32)
                    return c

                lax.fori_loop(0, NBLK, blk_body, 0)

        def add_recv(acc_ref, slot):
            def blk_body(b, c):
                sl = pl.ds(b * BLK, BLK)
                acc_ref[sl, :] = (acc_ref[sl, :]
                                  + rs_buf[slot, sl, :].astype(jnp.float32))
                return c
            lax.fori_loop(0, NBLK, blk_body, 0)

        def to_bf16(dst_ref, src_ref):
            def blk_body(b, c):
                sl = pl.ds(b * BLK, BLK)
                dst_ref[sl, :] = src_ref[sl, :].astype(jnp.bfloat16)
                return c
            lax.fori_loop(0, NBLK, blk_body, 0)

        def rcopy(src, dst, ssem, rsem):
            return pltpu.make_async_remote_copy(
                src_ref=src, dst_ref=dst, send_sem=ssem, recv_sem=rsem,
                device_id=(right,), device_id_type=pl.DeviceIdType.MESH)

        to_bf16(rs_send, x_ref)
        x0 = rcopy(rs_send, ag_x.at[0], ag_x_send.at[0], ag_x_recv.at[0])
        m0 = rcopy(meta_local, ag_meta.at[0], ag_m_send.at[0], ag_m_recv.at[0])
        x0.start()
        m0.start()
        accumulate(x_ref, my_i0, my_i1, my_g0, my_g1, out_ref)
        x0.wait()
        m0.wait()

        x1 = rcopy(ag_x.at[0], ag_x.at[1], ag_x_send.at[1], ag_x_recv.at[1])
        m1 = rcopy(ag_meta.at[0], ag_meta.at[1], ag_m_send.at[1], ag_m_recv.at[1])
        x1.start()
        m1.start()
        i0, i1, gg0, gg1 = meta_cols(ag_meta.at[0])
        accumulate(ag_x.at[0], i0, i1, gg0, gg1, p_acc)
        x1.wait()
        m1.wait()
        pl.semaphore_signal(credit_ag, inc=1, device_id=(left,),
                            device_id_type=pl.DeviceIdType.MESH)

        to_bf16(rs_send, p_acc)
        r0 = rcopy(rs_send, rs_buf.at[0], rs_send_sem.at[0], rs_recv_sem.at[0])
        r0.start()
        pl.semaphore_wait(credit_ag, 1)
        x2 = rcopy(ag_x.at[1], ag_x.at[0], ag_x_send.at[2], ag_x_recv.at[2])
        m2 = rcopy(ag_meta.at[1], ag_meta.at[0], ag_m_send.at[2], ag_m_recv.at[2])
        x2.start()
        m2.start()
        i0, i1, gg0, gg1 = meta_cols(ag_meta.at[1])
        accumulate(ag_x.at[1], i0, i1, gg0, gg1, p_acc)
        r0.wait()
        add_recv(p_acc, 0)
        pl.semaphore_signal(credit_rs, inc=1, device_id=(left,),
                            device_id_type=pl.DeviceIdType.MESH)

        to_bf16(rs_send, p_acc)
        r1 = rcopy(rs_send, rs_buf.at[1], rs_send_sem.at[1], rs_recv_sem.at[1])
        r1.start()
        x2.wait()
        m2.wait()
        i0, i1, gg0, gg1 = meta_cols(ag_meta.at[0])
        accumulate(ag_x.at[0], i0, i1, gg0, gg1, p_acc)
        r1.wait()
        add_recv(p_acc, 1)

        to_bf16(rs_send, p_acc)
        pl.semaphore_wait(credit_rs, 1)
        r2 = rcopy(rs_send, rs_buf.at[0], rs_send_sem.at[2], rs_recv_sem.at[2])
        r2.start()
        r2.wait()
        add_recv(out_ref, 0)

    return pl.pallas_call(
        body,
        out_shape=jax.ShapeDtypeStruct((TOK, D), jnp.float32),
        in_specs=[
            pl.BlockSpec(memory_space=pltpu.MemorySpace.VMEM),
            pl.BlockSpec(memory_space=pltpu.MemorySpace.VMEM),
            pl.BlockSpec(memory_space=pltpu.MemorySpace.VMEM),
            pl.BlockSpec(memory_space=pltpu.MemorySpace.HBM),
        ],
        out_specs=pl.BlockSpec(memory_space=pltpu.MemorySpace.VMEM),
        scratch_shapes=[
            pltpu.VMEM((8, TOK), jnp.float32),
            pltpu.VMEM((2, TOK, D), jnp.bfloat16),
            pltpu.VMEM((2, 8, TOK), jnp.float32),
            pltpu.VMEM((TOK, D), jnp.float32),
            pltpu.VMEM((TOK, D), jnp.bfloat16),
            pltpu.VMEM((2, TOK, D), jnp.bfloat16),
            pltpu.VMEM((2, D, D), jnp.float32),
            pltpu.VMEM((TOK, 1), jnp.float32),
            pltpu.SemaphoreType.DMA((3,)),
            pltpu.SemaphoreType.DMA((3,)),
            pltpu.SemaphoreType.DMA((3,)),
            pltpu.SemaphoreType.DMA((3,)),
            pltpu.SemaphoreType.DMA((3,)),
            pltpu.SemaphoreType.DMA((3,)),
            pltpu.SemaphoreType.DMA((2,)),
            pltpu.SemaphoreType.REGULAR,
            pltpu.SemaphoreType.REGULAR,
        ],
        compiler_params=pltpu.CompilerParams(
            collective_id=0,
            vmem_limit_bytes=64 * 1024 * 1024,
            has_side_effects=True,
        ),
    )(x, router_W, route_idx, expert_W)


# device time: 251222 ns/iter; 1.8348x vs baseline; 1.2519x over previous
import jax
import jax.numpy as jnp
from jax import lax
from jax.experimental import pallas as pl
from jax.experimental.pallas import tpu as pltpu

N_DEV = 4
TOK = 2048
HTOK = TOK // 2
D = 1024
E_LOCAL = 8
N_EXP = 32
BLK = 512
NBLK = TOK // BLK
HBLK = HTOK // BLK


def kernel(x, router_W, route_idx, expert_W):
    def body(x_ref, rw_ref, idx_ref, ew_ref, out_ref,
             meta_r, meta_l, ag_x_r, ag_x_l, ag_m_r, ag_m_l,
             pacc_r, pacc_l, rssend_r, rssend_l, rsbuf_r, rsbuf_l,
             w_buf, gate_ref,
             agx_s_r, agx_r_r, agm_s_r, agm_r_r, rs_s_r, rs_r_r,
             agx_s_l, agx_r_l, agm_s_l, agm_r_l, rs_s_l, rs_r_l,
             w_sem, cr_ag_r, cr_ag_l, cr_rs_r, cr_rs_l):
        me = lax.axis_index("i")
        left = lax.rem(me + N_DEV - 1, N_DEV)
        right = lax.rem(me + 1, N_DEV)

        barrier = pltpu.get_barrier_semaphore()
        for nbr in (left, right):
            pl.semaphore_signal(barrier, inc=1, device_id=(nbr,),
                                device_id_type=pl.DeviceIdType.MESH)
        pl.semaphore_wait(barrier, 2)

        scores = jnp.dot(x_ref[...], rw_ref[...],
                         preferred_element_type=jnp.float32)
        smax = jnp.max(scores, axis=1, keepdims=True)
        probs = jnp.exp(scores - smax)
        probs = probs / jnp.sum(probs, axis=1, keepdims=True)
        idx = idx_ref[...]
        iota = lax.broadcasted_iota(jnp.int32, (TOK, N_EXP), 1)
        g0 = jnp.sum(jnp.where(iota == idx[:, 0:1], probs, 0.0),
                     axis=1, keepdims=True)
        g1 = jnp.sum(jnp.where(iota == idx[:, 1:2], probs, 0.0),
                     axis=1, keepdims=True)
        gs = g0 + g1
        my_g0 = g0 / gs
        my_g1 = g1 / gs
        my_i0 = idx[:, 0:1].astype(jnp.float32)
        my_i1 = idx[:, 1:2].astype(jnp.float32)
        meta_all = jnp.concatenate([my_g0, my_g1, my_i0, my_i1], axis=1)
        meta_r[0:4, :] = meta_all[0:HTOK, :].T
        meta_l[0:4, :] = meta_all[HTOK:TOK, :].T

        def meta_cols(meta_ref):
            mt = meta_ref[...].T
            return mt[:, 2:3], mt[:, 3:4], mt[:, 0:1], mt[:, 1:2]

        def w_start(e, slot):
            pltpu.make_async_copy(ew_ref.at[e], w_buf.at[slot],
                                  w_sem.at[slot]).start()

        def w_wait(e, slot):
            pltpu.make_async_copy(ew_ref.at[e], w_buf.at[slot],
                                  w_sem.at[slot]).wait()

        me_f = me.astype(jnp.float32)

        def gate_of(e, i0, i1, gg0, gg1):
            gid = me_f * E_LOCAL + e
            return (jnp.where(i0 == gid, gg0, 0.0)
                    + jnp.where(i1 == gid, gg1, 0.0))

        def accumulate_own(acc_ref):
            acc_ref[...] = jnp.zeros((TOK, D), jnp.float32)
            w_start(0, 0)
            for e in range(E_LOCAL):
                slot = e % 2
                if e + 1 < E_LOCAL:
                    w_start(e + 1, 1 - slot)
                gate_ref[...] = gate_of(e, my_i0, my_i1, my_g0, my_g1)
                w_wait(e, slot)

                def blk_body(b, c):
                    sl = pl.ds(b * BLK, BLK)
                    xm = x_ref[sl, :] * gate_ref[sl, :]
                    acc_ref[sl, :] = acc_ref[sl, :] + jnp.dot(
                        xm, w_buf[slot, :, :],
                        preferred_element_type=jnp.float32)
                    return c

                lax.fori_loop(0, NBLK, blk_body, 0)

        def accumulate2(xa, cols_a, acc_a, xb, cols_b, acc_b):
            ia0, ia1, ga0, ga1 = cols_a
            ib0, ib1, gb0, gb1 = cols_b
            acc_a[...] = jnp.zeros((HTOK, D), jnp.float32)
            acc_b[...] = jnp.zeros((HTOK, D), jnp.float32)
            w_start(0, 0)
            for e in range(E_LOCAL):
                slot = e % 2
                if e + 1 < E_LOCAL:
                    w_start(e + 1, 1 - slot)
                gate_ref[0:HTOK, :] = gate_of(e, ia0, ia1, ga0, ga1)
                gate_ref[HTOK:TOK, :] = gate_of(e, ib0, ib1, gb0, gb1)
                w_wait(e, slot)

                def blk_a(b, c):
                    sl = pl.ds(b * BLK, BLK)
                    xm = xa[sl, :].astype(jnp.float32) * gate_ref[sl, :]
                    acc_a[sl, :] = acc_a[sl, :] + jnp.dot(
                        xm, w_buf[slot, :, :],
                        preferred_element_type=jnp.float32)
                    return c

                def blk_b(b, c):
                    sl = pl.ds(b * BLK, BLK)
                    gsl = pl.ds(HTOK + b * BLK, BLK)
                    xm = xb[sl, :].astype(jnp.float32) * gate_ref[gsl, :]
                    acc_b[sl, :] = acc_b[sl, :] + jnp.dot(
                        xm, w_buf[slot, :, :],
                        preferred_element_type=jnp.float32)
                    return c

                lax.fori_loop(0, HBLK, blk_a, 0)
                lax.fori_loop(0, HBLK, blk_b, 0)

        def to_bf16(dst_ref, src_ref, row_off):
            def blk_body(b, c):
                dst_ref[pl.ds(b * BLK, BLK), :] = src_ref[
                    pl.ds(row_off + b * BLK, BLK), :].astype(jnp.bfloat16)
                return c
            lax.fori_loop(0, HBLK, blk_body, 0)

        def add_recv(acc_ref, rsbuf, slot, row_off):
            def blk_body(b, c):
                sl = pl.ds(row_off + b * BLK, BLK)
                acc_ref[sl, :] = (acc_ref[sl, :]
                                  + rsbuf[slot, pl.ds(b * BLK, BLK), :]
                                  .astype(jnp.float32))
                return c
            lax.fori_loop(0, HBLK, blk_body, 0)

        def rc(src, dst, ssem, rsem, dev):
            return pltpu.make_async_remote_copy(
                src_ref=src, dst_ref=dst, send_sem=ssem, recv_sem=rsem,
                device_id=(dev,), device_id_type=pl.DeviceIdType.MESH)

        def signal(sem, dev):
            pl.semaphore_signal(sem, inc=1, device_id=(dev,),
                                device_id_type=pl.DeviceIdType.MESH)

        to_bf16(rssend_r, x_ref, 0)
        to_bf16(rssend_l, x_ref, HTOK)
        x0r = rc(rssend_r, ag_x_r.at[0], agx_s_r.at[0], agx_r_r.at[0], right)
        m0r = rc(meta_r, ag_m_r.at[0], agm_s_r.at[0], agm_r_r.at[0], right)
        x0l = rc(rssend_l, ag_x_l.at[0], agx_s_l.at[0], agx_r_l.at[0], left)
        m0l = rc(meta_l, ag_m_l.at[0], agm_s_l.at[0], agm_r_l.at[0], left)
        for t in (x0r, m0r, x0l, m0l):
            t.start()
        accumulate_own(out_ref)
        for t in (x0r, m0r, x0l, m0l):
            t.wait()

        x1r = rc(ag_x_r.at[0], ag_x_r.at[1], agx_s_r.at[1], agx_r_r.at[1], right)
        m1r = rc(ag_m_r.at[0], ag_m_r.at[1], agm_s_r.at[1], agm_r_r.at[1], right)
        x1l = rc(ag_x_l.at[0], ag_x_l.at[1], agx_s_l.at[1], agx_r_l.at[1], left)
        m1l = rc(ag_m_l.at[0], ag_m_l.at[1], agm_s_l.at[1], agm_r_l.at[1], left)
        for t in (x1r, m1r, x1l, m1l):
            t.start()
        accumulate2(ag_x_r.at[0], meta_cols(ag_m_r.at[0]), pacc_r,
                    ag_x_l.at[0], meta_cols(ag_m_l.at[0]), pacc_l)
        for t in (x1r, m1r, x1l, m1l):
            t.wait()
        signal(cr_ag_r, left)
        signal(cr_ag_l, right)

        to_bf16(rssend_r, pacc_r, 0)
        to_bf16(rssend_l, pacc_l, 0)
        r0r = rc(rssend_r, rsbuf_r.at[0], rs_s_r.at[0], rs_r_r.at[0], right)
        r0l = rc(rssend_l, rsbuf_l.at[0], rs_s_l.at[0], rs_r_l.at[0], left)
        r0r.start()
        r0l.start()
        pl.semaphore_wait(cr_ag_r, 1)
        pl.semaphore_wait(cr_ag_l, 1)
        x2r = rc(ag_x_r.at[1], ag_x_r.at[0], agx_s_r.at[2], agx_r_r.at[2], right)
        m2r = rc(ag_m_r.at[1], ag_m_r.at[0], agm_s_r.at[2], agm_r_r.at[2], right)
        x2l = rc(ag_x_l.at[1], ag_x_l.at[0], agx_s_l.at[2], agx_r_l.at[2], left)
        m2l = rc(ag_m_l.at[1], ag_m_l.at[0], agm_s_l.at[2], agm_r_l.at[2], left)
        for t in (x2r, m2r, x2l, m2l):
            t.start()
        accumulate2(ag_x_r.at[1], meta_cols(ag_m_r.at[1]), pacc_r,
                    ag_x_l.at[1], meta_cols(ag_m_l.at[1]), pacc_l)
        r0r.wait()
        r0l.wait()
        add_recv(pacc_r, rsbuf_r, 0, 0)
        add_recv(pacc_l, rsbuf_l, 0, 0)
        signal(cr_rs_r, left)
        signal(cr_rs_l, right)

        to_bf16(rssend_r, pacc_r, 0)
        to_bf16(rssend_l, pacc_l, 0)
        r1r = rc(rssend_r, rsbuf_r.at[1], rs_s_r.at[1], rs_r_r.at[1], right)
        r1l = rc(rssend_l, rsbuf_l.at[1], rs_s_l.at[1], rs_r_l.at[1], left)
        r1r.start()
        r1l.start()
        for t in (x2r, m2r, x2l, m2l):
            t.wait()
        accumulate2(ag_x_r.at[0], meta_cols(ag_m_r.at[0]), pacc_r,
                    ag_x_l.at[0], meta_cols(ag_m_l.at[0]), pacc_l)
        r1r.wait()
        r1l.wait()
        add_recv(pacc_r, rsbuf_r, 1, 0)
        add_recv(pacc_l, rsbuf_l, 1, 0)

        to_bf16(rssend_r, pacc_r, 0)
        to_bf16(rssend_l, pacc_l, 0)
        pl.semaphore_wait(cr_rs_r, 1)
        pl.semaphore_wait(cr_rs_l, 1)
        r2r = rc(rssend_r, rsbuf_r.at[0], rs_s_r.at[2], rs_r_r.at[2], right)
        r2l = rc(rssend_l, rsbuf_l.at[0], rs_s_l.at[2], rs_r_l.at[2], left)
        r2r.start()
        r2l.start()
        r2r.wait()
        r2l.wait()
        add_recv(out_ref, rsbuf_r, 0, 0)
        add_recv(out_ref, rsbuf_l, 0, HTOK)

    return pl.pallas_call(
        body,
        out_shape=jax.ShapeDtypeStruct((TOK, D), jnp.float32),
        in_specs=[
            pl.BlockSpec(memory_space=pltpu.MemorySpace.VMEM),
            pl.BlockSpec(memory_space=pltpu.MemorySpace.VMEM),
            pl.BlockSpec(memory_space=pltpu.MemorySpace.VMEM),
            pl.BlockSpec(memory_space=pltpu.MemorySpace.HBM),
        ],
        out_specs=pl.BlockSpec(memory_space=pltpu.MemorySpace.VMEM),
        scratch_shapes=[
            pltpu.VMEM((8, HTOK), jnp.float32),
            pltpu.VMEM((8, HTOK), jnp.float32),
            pltpu.VMEM((2, HTOK, D), jnp.bfloat16),
            pltpu.VMEM((2, HTOK, D), jnp.bfloat16),
            pltpu.VMEM((2, 8, HTOK), jnp.float32),
            pltpu.VMEM((2, 8, HTOK), jnp.float32),
            pltpu.VMEM((HTOK, D), jnp.float32),
            pltpu.VMEM((HTOK, D), jnp.float32),
            pltpu.VMEM((HTOK, D), jnp.bfloat16),
            pltpu.VMEM((HTOK, D), jnp.bfloat16),
            pltpu.VMEM((2, HTOK, D), jnp.bfloat16),
            pltpu.VMEM((2, HTOK, D), jnp.bfloat16),
            pltpu.VMEM((2, D, D), jnp.float32),
            pltpu.VMEM((TOK, 1), jnp.float32),
            pltpu.SemaphoreType.DMA((3,)),
            pltpu.SemaphoreType.DMA((3,)),
            pltpu.SemaphoreType.DMA((3,)),
            pltpu.SemaphoreType.DMA((3,)),
            pltpu.SemaphoreType.DMA((3,)),
            pltpu.SemaphoreType.DMA((3,)),
            pltpu.SemaphoreType.DMA((3,)),
            pltpu.SemaphoreType.DMA((3,)),
            pltpu.SemaphoreType.DMA((3,)),
            pltpu.SemaphoreType.DMA((3,)),
            pltpu.SemaphoreType.DMA((3,)),
            pltpu.SemaphoreType.DMA((3,)),
            pltpu.SemaphoreType.DMA((2,)),
            pltpu.SemaphoreType.REGULAR,
            pltpu.SemaphoreType.REGULAR,
            pltpu.SemaphoreType.REGULAR,
            pltpu.SemaphoreType.REGULAR,
        ],
        compiler_params=pltpu.CompilerParams(
            collective_id=0,
            vmem_limit_bytes=64 * 1024 * 1024,
            has_side_effects=True,
        ),
    )(x, router_W, route_idx, expert_W)


# device time: 250701 ns/iter; 1.8386x vs baseline; 1.0021x over previous
import jax
import jax.numpy as jnp
from jax import lax
from jax.experimental import pallas as pl
from jax.experimental.pallas import tpu as pltpu

N_DEV = 4
TOK = 2048
HTOK = TOK // 2
D = 1024
E_LOCAL = 8
N_EXP = 32
BLK = 512
NBLK = TOK // BLK
HBLK = HTOK // BLK


def kernel(x, router_W, route_idx, expert_W):
    def body(x_ref, rw_ref, idx_ref, ew_ref, out_ref,
             meta_r, meta_l, ag_x_r, ag_x_l, ag_m_r, ag_m_l,
             pacc_r, pacc_l, rssend_r, rssend_l, rsbuf_r, rsbuf_l,
             w_buf, gate_ref,
             agx_s_r, agx_r_r, agm_s_r, agm_r_r, rs_s_r, rs_r_r,
             agx_s_l, agx_r_l, agm_s_l, agm_r_l, rs_s_l, rs_r_l,
             w_sem, cr_ag_r, cr_ag_l, cr_rs_r, cr_rs_l):
        me = lax.axis_index("i")
        left = lax.rem(me + N_DEV - 1, N_DEV)
        right = lax.rem(me + 1, N_DEV)

        barrier = pltpu.get_barrier_semaphore()
        for nbr in (left, right):
            pl.semaphore_signal(barrier, inc=1, device_id=(nbr,),
                                device_id_type=pl.DeviceIdType.MESH)
        pl.semaphore_wait(barrier, 2)

        def to_bf16_early(dst_ref, src_ref, row_off):
            def blk_body(b, c):
                dst_ref[pl.ds(b * BLK, BLK), :] = src_ref[
                    pl.ds(row_off + b * BLK, BLK), :].astype(jnp.bfloat16)
                return c
            lax.fori_loop(0, HBLK, blk_body, 0)

        def rc_early(src, dst, ssem, rsem, dev):
            return pltpu.make_async_remote_copy(
                src_ref=src, dst_ref=dst, send_sem=ssem, recv_sem=rsem,
                device_id=(dev,), device_id_type=pl.DeviceIdType.MESH)

        to_bf16_early(rssend_r, x_ref, 0)
        to_bf16_early(rssend_l, x_ref, HTOK)
        x0r = rc_early(rssend_r, ag_x_r.at[0], agx_s_r.at[0], agx_r_r.at[0],
                       right)
        x0l = rc_early(rssend_l, ag_x_l.at[0], agx_s_l.at[0], agx_r_l.at[0],
                       left)
        x0r.start()
        x0l.start()

        scores = jnp.dot(x_ref[...], rw_ref[...],
                         preferred_element_type=jnp.float32)
        smax = jnp.max(scores, axis=1, keepdims=True)
        probs = jnp.exp(scores - smax)
        probs = probs / jnp.sum(probs, axis=1, keepdims=True)
        idx = idx_ref[...]
        iota = lax.broadcasted_iota(jnp.int32, (TOK, N_EXP), 1)
        g0 = jnp.sum(jnp.where(iota == idx[:, 0:1], probs, 0.0),
                     axis=1, keepdims=True)
        g1 = jnp.sum(jnp.where(iota == idx[:, 1:2], probs, 0.0),
                     axis=1, keepdims=True)
        gs = g0 + g1
        my_g0 = g0 / gs
        my_g1 = g1 / gs
        my_i0 = idx[:, 0:1].astype(jnp.float32)
        my_i1 = idx[:, 1:2].astype(jnp.float32)
        meta_all = jnp.concatenate([my_g0, my_g1, my_i0, my_i1], axis=1)
        meta_r[0:4, :] = meta_all[0:HTOK, :].T
        meta_l[0:4, :] = meta_all[HTOK:TOK, :].T

        def meta_cols(meta_ref):
            mt = meta_ref[...].T
            return mt[:, 2:3], mt[:, 3:4], mt[:, 0:1], mt[:, 1:2]

        def w_start(e, slot):
            pltpu.make_async_copy(ew_ref.at[e], w_buf.at[slot],
                                  w_sem.at[slot]).start()

        def w_wait(e, slot):
            pltpu.make_async_copy(ew_ref.at[e], w_buf.at[slot],
                                  w_sem.at[slot]).wait()

        me_f = me.astype(jnp.float32)

        def gate_of(e, i0, i1, gg0, gg1):
            gid = me_f * E_LOCAL + e
            return (jnp.where(i0 == gid, gg0, 0.0)
                    + jnp.where(i1 == gid, gg1, 0.0))

        def accumulate_own(acc_ref):
            acc_ref[...] = jnp.zeros((TOK, D), jnp.float32)
            w_start(0, 0)
            for e in range(E_LOCAL):
                slot = e % 2
                if e + 1 < E_LOCAL:
                    w_start(e + 1, 1 - slot)
                gate_ref[...] = gate_of(e, my_i0, my_i1, my_g0, my_g1)
                w_wait(e, slot)

                def blk_body(b, c):
                    sl = pl.ds(b * BLK, BLK)
                    xm = x_ref[sl, :] * gate_ref[sl, :]
                    acc_ref[sl, :] = acc_ref[sl, :] + jnp.dot(
                        xm, w_buf[slot, :, :],
                        preferred_element_type=jnp.float32)
                    return c

                lax.fori_loop(0, NBLK, blk_body, 0)

        def accumulate2(xa, cols_a, acc_a, xb, cols_b, acc_b):
            ia0, ia1, ga0, ga1 = cols_a
            ib0, ib1, gb0, gb1 = cols_b
            acc_a[...] = jnp.zeros((HTOK, D), jnp.float32)
            acc_b[...] = jnp.zeros((HTOK, D), jnp.float32)
            w_start(0, 0)
            for e in range(E_LOCAL):
                slot = e % 2
                if e + 1 < E_LOCAL:
                    w_start(e + 1, 1 - slot)
                gate_ref[0:HTOK, :] = gate_of(e, ia0, ia1, ga0, ga1)
                gate_ref[HTOK:TOK, :] = gate_of(e, ib0, ib1, gb0, gb1)
                w_wait(e, slot)

                def blk_a(b, c):
                    sl = pl.ds(b * BLK, BLK)
                    xm = xa[sl, :].astype(jnp.float32) * gate_ref[sl, :]
                    acc_a[sl, :] = acc_a[sl, :] + jnp.dot(
                        xm, w_buf[slot, :, :],
                        preferred_element_type=jnp.float32)
                    return c

                def blk_b(b, c):
                    sl = pl.ds(b * BLK, BLK)
                    gsl = pl.ds(HTOK + b * BLK, BLK)
                    xm = xb[sl, :].astype(jnp.float32) * gate_ref[gsl, :]
                    acc_b[sl, :] = acc_b[sl, :] + jnp.dot(
                        xm, w_buf[slot, :, :],
                        preferred_element_type=jnp.float32)
                    return c

                lax.fori_loop(0, HBLK, blk_a, 0)
                lax.fori_loop(0, HBLK, blk_b, 0)

        def to_bf16(dst_ref, src_ref, row_off):
            def blk_body(b, c):
                dst_ref[pl.ds(b * BLK, BLK), :] = src_ref[
                    pl.ds(row_off + b * BLK, BLK), :].astype(jnp.bfloat16)
                return c
            lax.fori_loop(0, HBLK, blk_body, 0)

        def add_recv(acc_ref, rsbuf, slot, row_off):
            def blk_body(b, c):
                sl = pl.ds(row_off + b * BLK, BLK)
                acc_ref[sl, :] = (acc_ref[sl, :]
                                  + rsbuf[slot, pl.ds(b * BLK, BLK), :]
                                  .astype(jnp.float32))
                return c
            lax.fori_loop(0, HBLK, blk_body, 0)

        def rc(src, dst, ssem, rsem, dev):
            return pltpu.make_async_remote_copy(
                src_ref=src, dst_ref=dst, send_sem=ssem, recv_sem=rsem,
                device_id=(dev,), device_id_type=pl.DeviceIdType.MESH)

        def signal(sem, dev):
            pl.semaphore_signal(sem, inc=1, device_id=(dev,),
                                device_id_type=pl.DeviceIdType.MESH)

        m0r = rc(meta_r, ag_m_r.at[0], agm_s_r.at[0], agm_r_r.at[0], right)
        m0l = rc(meta_l, ag_m_l.at[0], agm_s_l.at[0], agm_r_l.at[0], left)
        m0r.start()
        m0l.start()
        accumulate_own(out_ref)
        for t in (x0r, m0r, x0l, m0l):
            t.wait()

        x1r = rc(ag_x_r.at[0], ag_x_r.at[1], agx_s_r.at[1], agx_r_r.at[1], right)
        m1r = rc(ag_m_r.at[0], ag_m_r.at[1], agm_s_r.at[1], agm_r_r.at[1], right)
        x1l = rc(ag_x_l.at[0], ag_x_l.at[1], agx_s_l.at[1], agx_r_l.at[1], left)
        m1l = rc(ag_m_l.at[0], ag_m_l.at[1], agm_s_l.at[1], agm_r_l.at[1], left)
        for t in (x1r, m1r, x1l, m1l):
            t.start()
        accumulate2(ag_x_r.at[0], meta_cols(ag_m_r.at[0]), pacc_r,
                    ag_x_l.at[0], meta_cols(ag_m_l.at[0]), pacc_l)
        for t in (x1r, m1r, x1l, m1l):
            t.wait()
        signal(cr_ag_r, left)
        signal(cr_ag_l, right)

        to_bf16(rssend_r, pacc_r, 0)
        to_bf16(rssend_l, pacc_l, 0)
        r0r = rc(rssend_r, rsbuf_r.at[0], rs_s_r.at[0], rs_r_r.at[0], right)
        r0l = rc(rssend_l, rsbuf_l.at[0], rs_s_l.at[0], rs_r_l.at[0], left)
        r0r.start()
        r0l.start()
        pl.semaphore_wait(cr_ag_r, 1)
        pl.semaphore_wait(cr_ag_l, 1)
        x2r = rc(ag_x_r.at[1], ag_x_r.at[0], agx_s_r.at[2], agx_r_r.at[2], right)
        m2r = rc(ag_m_r.at[1], ag_m_r.at[0], agm_s_r.at[2], agm_r_r.at[2], right)
        x2l = rc(ag_x_l.at[1], ag_x_l.at[0], agx_s_l.at[2], agx_r_l.at[2], left)
        m2l = rc(ag_m_l.at[1], ag_m_l.at[0], agm_s_l.at[2], agm_r_l.at[2], left)
        for t in (x2r, m2r, x2l, m2l):
            t.start()
        accumulate2(ag_x_r.at[1], meta_cols(ag_m_r.at[1]), pacc_r,
                    ag_x_l.at[1], meta_cols(ag_m_l.at[1]), pacc_l)
        r0r.wait()
        r0l.wait()
        add_recv(pacc_r, rsbuf_r, 0, 0)
        add_recv(pacc_l, rsbuf_l, 0, 0)
        signal(cr_rs_r, left)
        signal(cr_rs_l, right)

        to_bf16(rssend_r, pacc_r, 0)
        to_bf16(rssend_l, pacc_l, 0)
        r1r = rc(rssend_r, rsbuf_r.at[1], rs_s_r.at[1], rs_r_r.at[1], right)
        r1l = rc(rssend_l, rsbuf_l.at[1], rs_s_l.at[1], rs_r_l.at[1], left)
        r1r.start()
        r1l.start()
        for t in (x2r, m2r, x2l, m2l):
            t.wait()
        accumulate2(ag_x_r.at[0], meta_cols(ag_m_r.at[0]), pacc_r,
                    ag_x_l.at[0], meta_cols(ag_m_l.at[0]), pacc_l)
        r1r.wait()
        r1l.wait()
        add_recv(pacc_r, rsbuf_r, 1, 0)
        add_recv(pacc_l, rsbuf_l, 1, 0)

        to_bf16(rssend_r, pacc_r, 0)
        to_bf16(rssend_l, pacc_l, 0)
        pl.semaphore_wait(cr_rs_r, 1)
        pl.semaphore_wait(cr_rs_l, 1)
        r2r = rc(rssend_r, rsbuf_r.at[0], rs_s_r.at[2], rs_r_r.at[2], right)
        r2l = rc(rssend_l, rsbuf_l.at[0], rs_s_l.at[2], rs_r_l.at[2], left)
        r2r.start()
        r2l.start()
        r2r.wait()
        r2l.wait()
        add_recv(out_ref, rsbuf_r, 0, 0)
        add_recv(out_ref, rsbuf_l, 0, HTOK)

    return pl.pallas_call(
        body,
        out_shape=jax.ShapeDtypeStruct((TOK, D), jnp.float32),
        in_specs=[
            pl.BlockSpec(memory_space=pltpu.MemorySpace.VMEM),
            pl.BlockSpec(memory_space=pltpu.MemorySpace.VMEM),
            pl.BlockSpec(memory_space=pltpu.MemorySpace.VMEM),
            pl.BlockSpec(memory_space=pltpu.MemorySpace.HBM),
        ],
        out_specs=pl.BlockSpec(memory_space=pltpu.MemorySpace.VMEM),
        scratch_shapes=[
            pltpu.VMEM((8, HTOK), jnp.float32),
            pltpu.VMEM((8, HTOK), jnp.float32),
            pltpu.VMEM((2, HTOK, D), jnp.bfloat16),
            pltpu.VMEM((2, HTOK, D), jnp.bfloat16),
            pltpu.VMEM((2, 8, HTOK), jnp.float32),
            pltpu.VMEM((2, 8, HTOK), jnp.float32),
            pltpu.VMEM((HTOK, D), jnp.float32),
            pltpu.VMEM((HTOK, D), jnp.float32),
            pltpu.VMEM((HTOK, D), jnp.bfloat16),
            pltpu.VMEM((HTOK, D), jnp.bfloat16),
            pltpu.VMEM((2, HTOK, D), jnp.bfloat16),
            pltpu.VMEM((2, HTOK, D), jnp.bfloat16),
            pltpu.VMEM((2, D, D), jnp.float32),
            pltpu.VMEM((TOK, 1), jnp.float32),
            pltpu.SemaphoreType.DMA((3,)),
            pltpu.SemaphoreType.DMA((3,)),
            pltpu.SemaphoreType.DMA((3,)),
            pltpu.SemaphoreType.DMA((3,)),
            pltpu.SemaphoreType.DMA((3,)),
            pltpu.SemaphoreType.DMA((3,)),
            pltpu.SemaphoreType.DMA((3,)),
            pltpu.SemaphoreType.DMA((3,)),
            pltpu.SemaphoreType.DMA((3,)),
            pltpu.SemaphoreType.DMA((3,)),
            pltpu.SemaphoreType.DMA((3,)),
            pltpu.SemaphoreType.DMA((3,)),
            pltpu.SemaphoreType.DMA((2,)),
            pltpu.SemaphoreType.REGULAR,
            pltpu.SemaphoreType.REGULAR,
            pltpu.SemaphoreType.REGULAR,
            pltpu.SemaphoreType.REGULAR,
        ],
        compiler_params=pltpu.CompilerParams(
            collective_id=0,
            vmem_limit_bytes=64 * 1024 * 1024,
            has_side_effects=True,
        ),
    )(x, router_W, route_idx, expert_W)
